# baseline (speedup 1.0000x reference)
import jax
import jax.numpy as jnp
from jax import lax
from jax.experimental import pallas as pl
from jax.experimental.pallas import tpu as pltpu
from jax.experimental.pallas import tpu_sc as plsc

_N_TOKENS = 32768
_EMBED_DIM = 768
_NUM_EXPERTS = 8
_TOP_K = 2
_BLOCK = 4096
_NW = 32
_CHUNK = _N_TOKENS // _NW
_LANES = 16
_WPB = _BLOCK // _CHUNK


def _logits_body(xa_ref, xb_ref, wa_ref, wb_ref, b_ref, epsT_ref,
                 gate_ref, noisy_ref, noisy3_ref):
    e = _NUM_EXPERTS
    logits = (jnp.dot(xa_ref[...], wa_ref[...], preferred_element_type=jnp.float32)
              + jnp.dot(xb_ref[...], wb_ref[...], preferred_element_type=jnp.float32)
              + b_ref[...])
    gate_ref[...] = logits[:, :e]

    lt = logits.T
    n_t = lt[e:, :]
    sp = jnp.maximum(n_t, 0.0) + jnp.log1p(jnp.exp(-jnp.abs(n_t)))
    noisy_t = lt[:e, :] + epsT_ref[...] * sp
    noisy_ref[...] = noisy_t.T
    for k in range(_WPB):
        noisy3_ref[k] = noisy_t[:, k * _CHUNK:(k + 1) * _CHUNK]


def _route_body(noisy3_hbm, w1_hbm, w2_hbm, i1_hbm, i2_hbm,
                nv_ref, w1_ref, w2_ref, i1_ref, i2_ref):
    wid = lax.axis_index("s") * 2 + lax.axis_index("c")
    base = wid * _CHUNK
    pltpu.sync_copy(noisy3_hbm.at[wid], nv_ref)

    def step(t, carry):
        off = t * _LANES
        best1 = nv_ref[pl.ds(off, _LANES)]
        idx1 = jnp.zeros((_LANES,), jnp.int32)
        best2 = jnp.full((_LANES,), -jnp.inf, jnp.float32)
        idx2 = jnp.zeros((_LANES,), jnp.int32)
        for ex in range(1, _NUM_EXPERTS):
            v = nv_ref[pl.ds(ex * _CHUNK + off, _LANES)]
            new1 = v > best1
            new2 = jnp.logical_and(v <= best1, v > best2)
            idx2 = jnp.where(new1, idx1, jnp.where(new2, ex, idx2))
            best2 = jnp.where(new1, best1, jnp.where(new2, v, best2))
            idx1 = jnp.where(new1, ex, idx1)
            best1 = jnp.where(new1, v, best1)
        e2 = jnp.exp(best2 - best1)
        w1 = 1.0 / (1.0 + e2)
        w1_ref[pl.ds(off, _LANES)] = w1
        w2_ref[pl.ds(off, _LANES)] = e2 * w1
        i1_ref[pl.ds(off, _LANES)] = idx1
        i2_ref[pl.ds(off, _LANES)] = idx2
        return carry

    lax.fori_loop(0, _CHUNK // _LANES, step, 0)
    pltpu.sync_copy(w1_ref, w1_hbm.at[pl.ds(base, _CHUNK)])
    pltpu.sync_copy(w2_ref, w2_hbm.at[pl.ds(base, _CHUNK)])
    pltpu.sync_copy(i1_ref, i1_hbm.at[pl.ds(base, _CHUNK)])
    pltpu.sync_copy(i2_ref, i2_hbm.at[pl.ds(base, _CHUNK)])


def kernel(hidden_states, Wg, bg, Wn, bn):
    n, d = hidden_states.shape
    e = Wg.shape[1]
    eps = jax.random.normal(jax.random.key(42), (n, e), dtype=jnp.float32)
    epsT = eps.T
    w16 = jnp.concatenate([Wg, Wn], axis=1)
    b16 = jnp.concatenate([bg, bn]).reshape(1, 2 * e)
    grid = (n // _BLOCK,)
    gate_logits, noisy_logits, noisy3 = pl.pallas_call(
        _logits_body,
        grid=grid,
        in_specs=[
            pl.BlockSpec((_BLOCK, d // 2), lambda i: (i, 0)),
            pl.BlockSpec((_BLOCK, d // 2), lambda i: (i, 1)),
            pl.BlockSpec((d // 2, 2 * e), lambda i: (0, 0)),
            pl.BlockSpec((d // 2, 2 * e), lambda i: (1, 0)),
            pl.BlockSpec((1, 2 * e), lambda i: (0, 0)),
            pl.BlockSpec((e, _BLOCK), lambda i: (0, i)),
        ],
        out_specs=[
            pl.BlockSpec((_BLOCK, e), lambda i: (i, 0)),
            pl.BlockSpec((_BLOCK, e), lambda i: (i, 0)),
            pl.BlockSpec((_WPB, e, _CHUNK), lambda i: (i, 0, 0)),
        ],
        out_shape=[
            jax.ShapeDtypeStruct((n, e), jnp.float32),
            jax.ShapeDtypeStruct((n, e), jnp.float32),
            jax.ShapeDtypeStruct((_NW, e, _CHUNK), jnp.float32),
        ],
    )(hidden_states, hidden_states, w16, w16, b16, epsT)

    route = pl.kernel(
        _route_body,
        out_type=(
            jax.ShapeDtypeStruct((n,), jnp.float32),
            jax.ShapeDtypeStruct((n,), jnp.float32),
            jax.ShapeDtypeStruct((n,), jnp.int32),
            jax.ShapeDtypeStruct((n,), jnp.int32),
        ),
        mesh=plsc.VectorSubcoreMesh(core_axis_name="c", subcore_axis_name="s"),
        scratch_types=[
            pltpu.VMEM((e * _CHUNK,), jnp.float32),
            pltpu.VMEM((_CHUNK,), jnp.float32),
            pltpu.VMEM((_CHUNK,), jnp.float32),
            pltpu.VMEM((_CHUNK,), jnp.int32),
            pltpu.VMEM((_CHUNK,), jnp.int32),
        ],
    )
    w1, w2, i1, i2 = route(noisy3.reshape(_NW, e * _CHUNK))
    routing_weights = jnp.stack([w1, w2], axis=1)
    selected_experts = jnp.stack([i1, i2], axis=1)
    return (routing_weights, selected_experts, noisy_logits, gate_logits)


# speedup vs baseline: 1.2851x; 1.2851x over previous
import jax
import jax.numpy as jnp
from jax import lax
from jax.experimental import pallas as pl

_N_TOKENS = 32768
_EMBED_DIM = 768
_NUM_EXPERTS = 8
_TOP_K = 2
_BLOCK = 4096


def _router_body(xa_ref, xb_ref, wa_ref, wb_ref, b_ref, epsT_ref,
                 gate_ref, noisy_ref, w1_ref, w2_ref, i1_ref, i2_ref):
    e = _NUM_EXPERTS
    logits = (jnp.dot(xa_ref[...], wa_ref[...], preferred_element_type=jnp.float32)
              + jnp.dot(xb_ref[...], wb_ref[...], preferred_element_type=jnp.float32)
              + b_ref[...])
    gate_ref[...] = logits[:, :e]

    lt = logits.T
    g_t = lt[:e, :]
    n_t = lt[e:, :]
    sp = jnp.maximum(n_t, 0.0) + jnp.log1p(jnp.exp(-jnp.abs(n_t)))
    noisy_t = g_t + epsT_ref[...] * sp
    noisy_ref[...] = noisy_t.T

    iota = lax.broadcasted_iota(jnp.int32, noisy_t.shape, 0)
    m1 = jnp.max(noisy_t, axis=0, keepdims=True)
    i1 = jnp.min(jnp.where(noisy_t == m1, iota, e), axis=0, keepdims=True)
    masked = jnp.where(iota == i1, -jnp.inf, noisy_t)
    m2 = jnp.max(masked, axis=0, keepdims=True)
    i2 = jnp.min(jnp.where(masked == m2, iota, e), axis=0, keepdims=True)
    e2 = jnp.exp(m2 - m1)
    denom = 1.0 + e2
    w1_ref[...] = 1.0 / denom
    w2_ref[...] = e2 / denom
    i1_ref[...] = i1
    i2_ref[...] = i2


def kernel(hidden_states, Wg, bg, Wn, bn):
    n, d = hidden_states.shape
    e = Wg.shape[1]
    eps = jax.random.normal(jax.random.key(42), (n, e), dtype=jnp.float32)
    epsT = eps.T
    w16 = jnp.concatenate([Wg, Wn], axis=1)
    b16 = jnp.concatenate([bg, bn]).reshape(1, 2 * e)
    grid = (n // _BLOCK,)
    outs = pl.pallas_call(
        _router_body,
        grid=grid,
        in_specs=[
            pl.BlockSpec((_BLOCK, d // 2), lambda i: (i, 0)),
            pl.BlockSpec((_BLOCK, d // 2), lambda i: (i, 1)),
            pl.BlockSpec((d // 2, 2 * e), lambda i: (0, 0)),
            pl.BlockSpec((d // 2, 2 * e), lambda i: (1, 0)),
            pl.BlockSpec((1, 2 * e), lambda i: (0, 0)),
            pl.BlockSpec((e, _BLOCK), lambda i: (0, i)),
        ],
        out_specs=[
            pl.BlockSpec((_BLOCK, e), lambda i: (i, 0)),
            pl.BlockSpec((_BLOCK, e), lambda i: (i, 0)),
            pl.BlockSpec((1, _BLOCK), lambda i: (0, i)),
            pl.BlockSpec((1, _BLOCK), lambda i: (0, i)),
            pl.BlockSpec((1, _BLOCK), lambda i: (0, i)),
            pl.BlockSpec((1, _BLOCK), lambda i: (0, i)),
        ],
        out_shape=[
            jax.ShapeDtypeStruct((n, e), jnp.float32),
            jax.ShapeDtypeStruct((n, e), jnp.float32),
            jax.ShapeDtypeStruct((1, n), jnp.float32),
            jax.ShapeDtypeStruct((1, n), jnp.float32),
            jax.ShapeDtypeStruct((1, n), jnp.int32),
            jax.ShapeDtypeStruct((1, n), jnp.int32),
        ],
    )(hidden_states, hidden_states, w16, w16, b16, epsT)
    gate_logits, noisy_logits, w1, w2, i1, i2 = outs
    routing_weights = jnp.concatenate([w1, w2], axis=0).T
    selected_experts = jnp.concatenate([i1, i2], axis=0).T
    return (routing_weights, selected_experts, noisy_logits, gate_logits)
